# Initial kernel scaffold; baseline (speedup 1.0000x reference)
import jax
import jax.numpy as jnp
from jax.experimental import pallas as pl


def kernel(x, adj):
    raise NotImplementedError("write your pallas kernel here")


# speedup vs baseline: 30.0896x; 30.0896x over previous
import functools

import jax
import jax.numpy as jnp
from jax.experimental import pallas as pl
from jax.experimental.pallas import tpu as pltpu

_ROWS = 8
_CH = 512


def _mp_kernel(adj_ref, xt_ref, out_ref):
    n = adj_ref.shape[1]
    d = xt_ref.shape[0]
    neg = jnp.float32(-jnp.inf)
    xt = xt_ref[...]
    cols = []
    for r in range(_ROWS):
        m = adj_ref[r : r + 1, :] > 0
        acc = jnp.full((d, _CH), neg, dtype=jnp.float32)
        for c in range(0, n, _CH):
            acc = jnp.maximum(
                acc, jnp.where(m[:, c : c + _CH], xt[:, c : c + _CH], neg)
            )
        mx = jnp.max(acc, axis=1, keepdims=True)
        cols.append(jnp.where(mx > neg, mx, 0.0))
    blk = jnp.concatenate(cols, axis=1)
    out_ref[...] = blk.T


@jax.jit
def kernel(x, adj):
    n, d = x.shape
    xt = x.T
    grid = (n // _ROWS,)
    out = pl.pallas_call(
        _mp_kernel,
        grid=grid,
        in_specs=[
            pl.BlockSpec((_ROWS, n), lambda i: (i, 0)),
            pl.BlockSpec((d, n), lambda i: (0, 0)),
        ],
        out_specs=pl.BlockSpec((_ROWS, d), lambda i: (i, 0)),
        out_shape=jax.ShapeDtypeStruct((n, d), jnp.float32),
        compiler_params=pltpu.CompilerParams(
            dimension_semantics=("arbitrary",),
        ),
    )(adj, xt)
    return out


# speedup vs baseline: 35.5529x; 1.1816x over previous
import functools

import jax
import jax.numpy as jnp
from jax.experimental import pallas as pl
from jax.experimental.pallas import tpu as pltpu

_ROWS = 8
_CH = 128


_GRP = 4


def _mp_kernel(adj_ref, xt_ref, out_ref):
    n = adj_ref.shape[1]
    d = xt_ref.shape[0]
    neg = jnp.float32(-jnp.inf)
    cols = []
    for r0 in range(0, _ROWS, _GRP):
        accs = [jnp.full((d, _CH), neg, dtype=jnp.float32) for _ in range(_GRP)]
        for c in range(0, n, _CH):
            xc = xt_ref[:, c : c + _CH]
            for g in range(_GRP):
                m = adj_ref[r0 + g : r0 + g + 1, c : c + _CH] > 0
                pen = jnp.where(m, 0.0, neg)
                accs[g] = jnp.maximum(accs[g], xc + pen)
        for g in range(_GRP):
            mx = jnp.max(accs[g], axis=1, keepdims=True)
            cols.append(jnp.where(mx > neg, mx, 0.0))
    blk = jnp.concatenate(cols, axis=1)
    out_ref[...] = blk.T


@jax.jit
def kernel(x, adj):
    n, d = x.shape
    xt = x.T
    grid = (n // _ROWS,)
    out = pl.pallas_call(
        _mp_kernel,
        grid=grid,
        in_specs=[
            pl.BlockSpec((_ROWS, n), lambda i: (i, 0)),
            pl.BlockSpec((d, n), lambda i: (0, 0)),
        ],
        out_specs=pl.BlockSpec((_ROWS, d), lambda i: (i, 0)),
        out_shape=jax.ShapeDtypeStruct((n, d), jnp.float32),
        compiler_params=pltpu.CompilerParams(
            dimension_semantics=("arbitrary",),
        ),
    )(adj, xt)
    return out


# speedup vs baseline: 35.8743x; 1.0090x over previous
import functools

import jax
import jax.numpy as jnp
from jax.experimental import pallas as pl
from jax.experimental.pallas import tpu as pltpu

_ROWS = 8
_CH = 128


def _mp_kernel(adj_ref, xt_ref, out_ref):
    n = adj_ref.shape[2]
    d = xt_ref.shape[0]
    neg = jnp.float32(-jnp.inf)
    accs = [jnp.full((d, _CH), neg, dtype=jnp.float32) for _ in range(_ROWS)]
    for c in range(0, n, _CH):
        xc = xt_ref[:, c : c + _CH]
        for r in range(_ROWS):
            m = adj_ref[0, r : r + 1, c : c + _CH] > 0
            pen = jnp.where(m, 0.0, neg)
            accs[r] = jnp.maximum(accs[r], xc + pen)
    cols = []
    for r in range(_ROWS):
        mx = jnp.max(accs[r], axis=1, keepdims=True)
        cols.append(jnp.where(mx > neg, mx, 0.0))
    out_ref[0] = jnp.concatenate(cols, axis=1)


@jax.jit
def kernel(x, adj):
    n, d = x.shape
    xt = x.T
    adj3 = adj.reshape(n // _ROWS, _ROWS, n)
    out = pl.pallas_call(
        _mp_kernel,
        grid=(n // _ROWS,),
        in_specs=[
            pl.BlockSpec((1, _ROWS, n), lambda i: (i, 0, 0)),
            pl.BlockSpec((d, n), lambda i: (0, 0)),
        ],
        out_specs=pl.BlockSpec((1, d, _ROWS), lambda i: (i, 0, 0)),
        out_shape=jax.ShapeDtypeStruct((n // _ROWS, d, _ROWS), jnp.float32),
        compiler_params=pltpu.CompilerParams(
            dimension_semantics=("parallel",),
        ),
    )(adj3, xt)
    return out.transpose(0, 2, 1).reshape(n, d)
